# baseline (speedup 1.0000x reference)
import jax
import jax.numpy as jnp
from jax.experimental import pallas as pl

B, S, H = 1, 2048, 1024
NH, NKH, HD = 16, 4, 64
E, K, F = 8, 2, 512
EPS = 1e-05
THETA = 500000.0

SB = 256
NTB = S // SB
SBK = 256
GROUP = NH // NKH
GW = GROUP * HD
SCALE = 0.125

_F32 = jnp.float32
_BF16 = jnp.bfloat16


def _bf(x):
    return x.astype(_BF16)


def _shift_up(x, s):
    return jnp.concatenate([x[:, s:], x[:, :s]], axis=1)


def _shift_dn(x, s):
    return jnp.concatenate([x[:, -s:], x[:, :-s]], axis=1)


def _rope_full(x, cos_t, sina_t, sinb_t):
    half = HD // 2
    return x * cos_t + _shift_up(x, half) * sina_t + _shift_dn(x, half) * sinb_t


def _pre_kernel(h_ref, ln1_ref, wq_ref, wk_ref, wv_ref,
                cos_ref, sina_ref, sinb_ref, q_ref, k_ref, v_ref):
    x = h_ref[...]
    var = jnp.mean(x * x, axis=1, keepdims=True)
    x = _bf(ln1_ref[...] * (x * jax.lax.rsqrt(var + EPS)))
    q = jnp.dot(x, wq_ref[...], preferred_element_type=_F32)
    k = jnp.dot(x, wk_ref[...], preferred_element_type=_F32)
    v = jnp.dot(x, wv_ref[...], preferred_element_type=_F32)
    cos = cos_ref[...]
    sina = sina_ref[...]
    sinb = sinb_ref[...]
    cos_q = jnp.concatenate([cos] * NH, axis=1)
    sina_q = jnp.concatenate([sina] * NH, axis=1)
    sinb_q = jnp.concatenate([sinb] * NH, axis=1)
    q_ref[...] = _bf(_rope_full(q, cos_q, sina_q, sinb_q))
    cos_k = jnp.concatenate([cos] * NKH, axis=1)
    sina_k = jnp.concatenate([sina] * NKH, axis=1)
    sinb_k = jnp.concatenate([sinb] * NKH, axis=1)
    kr = _bf(_rope_full(k, cos_k, sina_k, sinb_k))
    vb = _bf(v)
    for h in range(NKH):
        k_ref[h] = kr[:, h * HD:(h + 1) * HD]
        v_ref[h] = vb[:, h * HD:(h + 1) * HD]


def _attn_kernel(q_ref, k_ref, v_ref, o_ref):
    qb = pl.program_id(1)
    q4 = q_ref[...]
    qm = jnp.concatenate(
        [q4[:, j * HD:(j + 1) * HD] for j in range(GROUP)], axis=0)
    neg = jnp.finfo(_F32).min
    gsb = GROUP * SB

    def step(s, carry, vc):
        acc, m, l = carry
        m_new = jnp.maximum(m, jnp.max(s, axis=1, keepdims=True))
        alpha = jnp.exp(m - m_new)
        p = jnp.exp(s - m_new)
        acc = acc * alpha + jnp.dot(_bf(p), vc, preferred_element_type=_F32)
        l = l * alpha + jnp.sum(p, axis=1, keepdims=True)
        return acc, m_new, l

    def body(c, carry):
        kc = k_ref[0, pl.ds(c * SBK, SBK), :]
        vc = v_ref[0, pl.ds(c * SBK, SBK), :]
        s = jax.lax.dot_general(
            qm, kc, (((1,), (1,)), ((), ())),
            preferred_element_type=_F32) * SCALE
        return step(s, carry, vc)

    acc0 = jnp.zeros((gsb, HD), _F32)
    m0 = jnp.full((gsb, 1), neg, _F32)
    l0 = jnp.zeros((gsb, 1), _F32)
    carry = jax.lax.fori_loop(0, qb, body, (acc0, m0, l0))
    kc = k_ref[0, pl.ds(qb * SBK, SBK), :]
    vc = v_ref[0, pl.ds(qb * SBK, SBK), :]
    s = jax.lax.dot_general(
        qm, kc, (((1,), (1,)), ((), ())),
        preferred_element_type=_F32) * SCALE
    rows = jax.lax.broadcasted_iota(jnp.int32, s.shape, 0) & (SB - 1)
    cols = jax.lax.broadcasted_iota(jnp.int32, s.shape, 1)
    s = jnp.where(rows >= cols, s, neg)
    acc, m, l = step(s, carry, vc)
    o = acc / l
    o_ref[...] = _bf(jnp.concatenate(
        [o[j * SB:(j + 1) * SB, :] for j in range(GROUP)], axis=1))


def _post_kernel(ao_ref, wo_ref, h_ref, ln2_ref, gate_ref,
                 h2_ref, x2_ref, logits_ref):
    attn = jnp.dot(ao_ref[...], wo_ref[...], preferred_element_type=_F32)
    h2 = h_ref[...] + attn
    var = jnp.mean(h2 * h2, axis=1, keepdims=True)
    x2 = ln2_ref[...] * (h2 * jax.lax.rsqrt(var + EPS))
    h2_ref[...] = h2
    x2_ref[...] = _bf(x2)
    logits_ref[...] = jnp.dot(x2, gate_ref[...], preferred_element_type=_F32)


def _moe_kernel(x2_ref, logits_ref, wgu_ref, wd_ref, h2_ref, o_ref):
    e = pl.program_id(0)
    logits = logits_ref[...]
    col = jax.lax.broadcasted_iota(jnp.int32, logits.shape, 1)
    m1 = jnp.max(logits, axis=1, keepdims=True)
    a1 = jnp.min(jnp.where(logits == m1, col, E), axis=1, keepdims=True)
    masked = jnp.where(col == a1, -jnp.inf, logits)
    m2 = jnp.max(masked, axis=1, keepdims=True)
    a2 = jnp.min(jnp.where(masked == m2, col, E), axis=1, keepdims=True)
    t = jnp.exp(m2 - m1)
    w1 = 1.0 / (1.0 + t)
    w2 = t / (1.0 + t)
    w_e = jnp.where(a1 == e, w1, 0.0) + jnp.where(a2 == e, w2, 0.0)

    x = x2_ref[...]
    gu = jnp.dot(x, wgu_ref[0], preferred_element_type=_F32)
    g = gu[:, :F]
    u = gu[:, F:]
    act = (g * jax.lax.logistic(g)) * u
    d = jnp.dot(_bf(act), wd_ref[0], preferred_element_type=_F32)
    contrib = w_e * d

    @pl.when(e == 0)
    def _():
        o_ref[...] = h2_ref[...] + contrib

    @pl.when(e > 0)
    def _():
        o_ref[...] += contrib


@jax.jit
def _forward_impl(h3, ln1_w, ln2_w, Wq, Wk, Wv, Wo, gate_w, Wg, Wu, Wd):
    h = h3.reshape(S, H)
    pos = jnp.arange(S, dtype=_F32)
    inv_freq = 1.0 / (THETA ** (jnp.arange(0, HD, 2, dtype=_F32) / HD))
    freqs = pos[:, None] * inv_freq[None, :]
    emb = jnp.concatenate([freqs, freqs], axis=-1)
    cos = jnp.cos(emb)
    sin = jnp.sin(emb)
    half = HD // 2
    lane = jnp.arange(HD)
    sina = jnp.where(lane < half, -sin, 0.0)
    sinb = jnp.where(lane >= half, sin, 0.0)

    q, k, v = pl.pallas_call(
        _pre_kernel,
        grid=(NTB,),
        in_specs=[
            pl.BlockSpec((SB, H), lambda i: (i, 0)),
            pl.BlockSpec((1, H), lambda i: (0, 0)),
            pl.BlockSpec((H, NH * HD), lambda i: (0, 0)),
            pl.BlockSpec((H, NKH * HD), lambda i: (0, 0)),
            pl.BlockSpec((H, NKH * HD), lambda i: (0, 0)),
            pl.BlockSpec((SB, HD), lambda i: (i, 0)),
            pl.BlockSpec((SB, HD), lambda i: (i, 0)),
            pl.BlockSpec((SB, HD), lambda i: (i, 0)),
        ],
        out_specs=[
            pl.BlockSpec((SB, NH * HD), lambda i: (i, 0)),
            pl.BlockSpec((NKH, SB, HD), lambda i: (0, i, 0)),
            pl.BlockSpec((NKH, SB, HD), lambda i: (0, i, 0)),
        ],
        out_shape=[
            jax.ShapeDtypeStruct((S, NH * HD), _BF16),
            jax.ShapeDtypeStruct((NKH, S, HD), _BF16),
            jax.ShapeDtypeStruct((NKH, S, HD), _BF16),
        ],
    )(h, ln1_w.reshape(1, H), _bf(Wq), _bf(Wk), _bf(Wv), cos, sina, sinb)

    o = pl.pallas_call(
        _attn_kernel,
        grid=(NKH, NTB),
        in_specs=[
            pl.BlockSpec((SB, GW), lambda g, i: (i, g)),
            pl.BlockSpec((1, S, HD), lambda g, i: (g, 0, 0)),
            pl.BlockSpec((1, S, HD), lambda g, i: (g, 0, 0)),
        ],
        out_specs=pl.BlockSpec((SB, GW), lambda g, i: (i, g)),
        out_shape=jax.ShapeDtypeStruct((S, NH * HD), _BF16),
    )(q, k, v)

    h2, x2, logits = pl.pallas_call(
        _post_kernel,
        grid=(NTB,),
        in_specs=[
            pl.BlockSpec((SB, NH * HD), lambda i: (i, 0)),
            pl.BlockSpec((NH * HD, H), lambda i: (0, 0)),
            pl.BlockSpec((SB, H), lambda i: (i, 0)),
            pl.BlockSpec((1, H), lambda i: (0, 0)),
            pl.BlockSpec((H, E), lambda i: (0, 0)),
        ],
        out_specs=[
            pl.BlockSpec((SB, H), lambda i: (i, 0)),
            pl.BlockSpec((SB, H), lambda i: (i, 0)),
            pl.BlockSpec((SB, E), lambda i: (i, 0)),
        ],
        out_shape=[
            jax.ShapeDtypeStruct((S, H), _F32),
            jax.ShapeDtypeStruct((S, H), _BF16),
            jax.ShapeDtypeStruct((S, E), _F32),
        ],
    )(o, _bf(Wo), h, ln2_w.reshape(1, H), gate_w)

    out = pl.pallas_call(
        _moe_kernel,
        grid=(E,),
        in_specs=[
            pl.BlockSpec((S, H), lambda e: (0, 0)),
            pl.BlockSpec((S, E), lambda e: (0, 0)),
            pl.BlockSpec((1, H, 2 * F), lambda e: (e, 0, 0)),
            pl.BlockSpec((1, F, H), lambda e: (e, 0, 0)),
            pl.BlockSpec((S, H), lambda e: (0, 0)),
        ],
        out_specs=pl.BlockSpec((S, H), lambda e: (0, 0)),
        out_shape=jax.ShapeDtypeStruct((S, H), _F32),
    )(x2, logits, _bf(jnp.concatenate([Wg, Wu], axis=2)), _bf(Wd), h2)

    return out.reshape(B, S, H)


def kernel(hidden_states, ln1_w, ln2_w, Wq, Wk, Wv, Wo, gate_w, Wg, Wu, Wd):
    return _forward_impl(hidden_states, ln1_w, ln2_w, Wq, Wk, Wv, Wo,
                         gate_w, Wg, Wu, Wd)


# speedup vs baseline: 1.0223x; 1.0223x over previous
import jax
import jax.numpy as jnp
from jax.experimental import pallas as pl

B, S, H = 1, 2048, 1024
NH, NKH, HD = 16, 4, 64
E, K, F = 8, 2, 512
EPS = 1e-05
THETA = 500000.0

SB = 256
NTB = S // SB
SBK = 256
GROUP = NH // NKH
GW = GROUP * HD
SCALE = 0.125

_F32 = jnp.float32
_BF16 = jnp.bfloat16


def _bf(x):
    return x.astype(_BF16)


def _shift_up(x, s):
    return jnp.concatenate([x[:, s:], x[:, :s]], axis=1)


def _shift_dn(x, s):
    return jnp.concatenate([x[:, -s:], x[:, :-s]], axis=1)


def _rope_full(x, cos_t, sina_t, sinb_t):
    half = HD // 2
    return x * cos_t + _shift_up(x, half) * sina_t + _shift_dn(x, half) * sinb_t


def _pre_kernel(h_ref, ln1_ref, wq_ref, wk_ref, wv_ref,
                cos_ref, sina_ref, sinb_ref, q_ref, k_ref, v_ref):
    x = h_ref[...]
    var = jnp.mean(x * x, axis=1, keepdims=True)
    x = _bf(ln1_ref[...] * (x * jax.lax.rsqrt(var + EPS)))
    q = jnp.dot(x, wq_ref[...], preferred_element_type=_F32)
    k = jnp.dot(x, wk_ref[...], preferred_element_type=_F32)
    v = jnp.dot(x, wv_ref[...], preferred_element_type=_F32)
    cos = cos_ref[...]
    sina = sina_ref[...]
    sinb = sinb_ref[...]
    cos_q = jnp.concatenate([cos] * NH, axis=1)
    sina_q = jnp.concatenate([sina] * NH, axis=1)
    sinb_q = jnp.concatenate([sinb] * NH, axis=1)
    q_ref[...] = _bf(_rope_full(q, cos_q, sina_q, sinb_q))
    cos_k = jnp.concatenate([cos] * NKH, axis=1)
    sina_k = jnp.concatenate([sina] * NKH, axis=1)
    sinb_k = jnp.concatenate([sinb] * NKH, axis=1)
    kr = _bf(_rope_full(k, cos_k, sina_k, sinb_k))
    vb = _bf(v)
    for h in range(NKH):
        k_ref[h] = kr[:, h * HD:(h + 1) * HD]
        v_ref[h] = vb[:, h * HD:(h + 1) * HD]


def _attn_kernel(q_ref, k_ref, v_ref, o_ref):
    qb = pl.program_id(1)
    q4 = q_ref[...]
    qm = jnp.concatenate(
        [q4[:, j * HD:(j + 1) * HD] for j in range(GROUP)], axis=0)
    neg = jnp.finfo(_F32).min
    gsb = GROUP * SB

    def step(s, carry, vc):
        acc, m, l = carry
        m_new = jnp.maximum(m, jnp.max(s, axis=1, keepdims=True))
        alpha = jnp.exp(m - m_new)
        p = jnp.exp(s - m_new)
        acc = acc * alpha + jnp.dot(_bf(p), vc, preferred_element_type=_F32)
        l = l * alpha + jnp.sum(p, axis=1, keepdims=True)
        return acc, m_new, l

    def body(c, carry):
        kc = k_ref[0, pl.ds(c * SBK, SBK), :]
        vc = v_ref[0, pl.ds(c * SBK, SBK), :]
        s = jax.lax.dot_general(
            qm, kc, (((1,), (1,)), ((), ())),
            preferred_element_type=_F32) * SCALE
        return step(s, carry, vc)

    acc0 = jnp.zeros((gsb, HD), _F32)
    m0 = jnp.full((gsb, 1), neg, _F32)
    l0 = jnp.zeros((gsb, 1), _F32)
    carry = jax.lax.fori_loop(0, qb, body, (acc0, m0, l0))
    kc = k_ref[0, pl.ds(qb * SBK, SBK), :]
    vc = v_ref[0, pl.ds(qb * SBK, SBK), :]
    s = jax.lax.dot_general(
        qm, kc, (((1,), (1,)), ((), ())),
        preferred_element_type=_F32) * SCALE
    rows = jax.lax.broadcasted_iota(jnp.int32, s.shape, 0) & (SB - 1)
    cols = jax.lax.broadcasted_iota(jnp.int32, s.shape, 1)
    s = jnp.where(rows >= cols, s, neg)
    acc, m, l = step(s, carry, vc)
    o = acc / l
    o_ref[...] = _bf(jnp.concatenate(
        [o[j * SB:(j + 1) * SB, :] for j in range(GROUP)], axis=1))


def _post_kernel(ao_ref, wo_ref, h_ref, ln2_ref, gate_ref,
                 h2_ref, x2_ref, logits_ref):
    attn = jnp.dot(ao_ref[...], wo_ref[...], preferred_element_type=_F32)
    h2 = h_ref[...] + attn
    var = jnp.mean(h2 * h2, axis=1, keepdims=True)
    x2 = ln2_ref[...] * (h2 * jax.lax.rsqrt(var + EPS))
    h2_ref[...] = h2
    x2_ref[...] = _bf(x2)
    logits_ref[...] = jnp.dot(x2, gate_ref[...], preferred_element_type=_F32)


EPG = 2
NEG = E // EPG
SM = 1024


def _moe_kernel(x2_ref, logits_ref, wgu_ref, wd_ref, h2_ref, o_ref):
    eg = pl.program_id(1)
    logits = logits_ref[...]
    col = jax.lax.broadcasted_iota(jnp.int32, logits.shape, 1)
    m1 = jnp.max(logits, axis=1, keepdims=True)
    a1 = jnp.min(jnp.where(logits == m1, col, E), axis=1, keepdims=True)
    masked = jnp.where(col == a1, -jnp.inf, logits)
    m2 = jnp.max(masked, axis=1, keepdims=True)
    a2 = jnp.min(jnp.where(masked == m2, col, E), axis=1, keepdims=True)
    t = jnp.exp(m2 - m1)
    w1 = 1.0 / (1.0 + t)
    w2 = t / (1.0 + t)

    x = x2_ref[...]
    total = None
    for j in range(EPG):
        e = eg * EPG + j
        w_e = jnp.where(a1 == e, w1, 0.0) + jnp.where(a2 == e, w2, 0.0)
        gu = jnp.dot(x, wgu_ref[e], preferred_element_type=_F32)
        g = gu[:, :F]
        u = gu[:, F:]
        act = _bf((g * jax.lax.logistic(g)) * u)
        d = jnp.dot(act, wd_ref[e], preferred_element_type=_F32)
        contrib = w_e * d
        total = contrib if total is None else total + contrib

    @pl.when(eg == 0)
    def _():
        o_ref[...] = h2_ref[...] + total

    @pl.when(eg > 0)
    def _():
        o_ref[...] += total


@jax.jit
def _forward_impl(h3, ln1_w, ln2_w, Wq, Wk, Wv, Wo, gate_w, Wg, Wu, Wd):
    h = h3.reshape(S, H)
    pos = jnp.arange(S, dtype=_F32)
    inv_freq = 1.0 / (THETA ** (jnp.arange(0, HD, 2, dtype=_F32) / HD))
    freqs = pos[:, None] * inv_freq[None, :]
    emb = jnp.concatenate([freqs, freqs], axis=-1)
    cos = jnp.cos(emb)
    sin = jnp.sin(emb)
    half = HD // 2
    lane = jnp.arange(HD)
    sina = jnp.where(lane < half, -sin, 0.0)
    sinb = jnp.where(lane >= half, sin, 0.0)

    q, k, v = pl.pallas_call(
        _pre_kernel,
        grid=(NTB,),
        in_specs=[
            pl.BlockSpec((SB, H), lambda i: (i, 0)),
            pl.BlockSpec((1, H), lambda i: (0, 0)),
            pl.BlockSpec((H, NH * HD), lambda i: (0, 0)),
            pl.BlockSpec((H, NKH * HD), lambda i: (0, 0)),
            pl.BlockSpec((H, NKH * HD), lambda i: (0, 0)),
            pl.BlockSpec((SB, HD), lambda i: (i, 0)),
            pl.BlockSpec((SB, HD), lambda i: (i, 0)),
            pl.BlockSpec((SB, HD), lambda i: (i, 0)),
        ],
        out_specs=[
            pl.BlockSpec((SB, NH * HD), lambda i: (i, 0)),
            pl.BlockSpec((NKH, SB, HD), lambda i: (0, i, 0)),
            pl.BlockSpec((NKH, SB, HD), lambda i: (0, i, 0)),
        ],
        out_shape=[
            jax.ShapeDtypeStruct((S, NH * HD), _BF16),
            jax.ShapeDtypeStruct((NKH, S, HD), _BF16),
            jax.ShapeDtypeStruct((NKH, S, HD), _BF16),
        ],
    )(h, ln1_w.reshape(1, H), _bf(Wq), _bf(Wk), _bf(Wv), cos, sina, sinb)

    o = pl.pallas_call(
        _attn_kernel,
        grid=(NKH, NTB),
        in_specs=[
            pl.BlockSpec((SB, GW), lambda g, i: (i, g)),
            pl.BlockSpec((1, S, HD), lambda g, i: (g, 0, 0)),
            pl.BlockSpec((1, S, HD), lambda g, i: (g, 0, 0)),
        ],
        out_specs=pl.BlockSpec((SB, GW), lambda g, i: (i, g)),
        out_shape=jax.ShapeDtypeStruct((S, NH * HD), _BF16),
    )(q, k, v)

    h2, x2, logits = pl.pallas_call(
        _post_kernel,
        grid=(NTB,),
        in_specs=[
            pl.BlockSpec((SB, NH * HD), lambda i: (i, 0)),
            pl.BlockSpec((NH * HD, H), lambda i: (0, 0)),
            pl.BlockSpec((SB, H), lambda i: (i, 0)),
            pl.BlockSpec((1, H), lambda i: (0, 0)),
            pl.BlockSpec((H, E), lambda i: (0, 0)),
        ],
        out_specs=[
            pl.BlockSpec((SB, H), lambda i: (i, 0)),
            pl.BlockSpec((SB, H), lambda i: (i, 0)),
            pl.BlockSpec((SB, E), lambda i: (i, 0)),
        ],
        out_shape=[
            jax.ShapeDtypeStruct((S, H), _F32),
            jax.ShapeDtypeStruct((S, H), _BF16),
            jax.ShapeDtypeStruct((S, E), _F32),
        ],
    )(o, _bf(Wo), h, ln2_w.reshape(1, H), gate_w)

    out = pl.pallas_call(
        _moe_kernel,
        grid=(S // SM, NEG),
        in_specs=[
            pl.BlockSpec((SM, H), lambda t, e: (t, 0)),
            pl.BlockSpec((SM, E), lambda t, e: (t, 0)),
            pl.BlockSpec((E, H, 2 * F), lambda t, e: (0, 0, 0)),
            pl.BlockSpec((E, F, H), lambda t, e: (0, 0, 0)),
            pl.BlockSpec((SM, H), lambda t, e: (t, 0)),
        ],
        out_specs=pl.BlockSpec((SM, H), lambda t, e: (t, 0)),
        out_shape=jax.ShapeDtypeStruct((S, H), _F32),
    )(x2, logits, _bf(jnp.concatenate([Wg, Wu], axis=2)), _bf(Wd), h2)

    return out.reshape(B, S, H)


def kernel(hidden_states, ln1_w, ln2_w, Wq, Wk, Wv, Wo, gate_w, Wg, Wu, Wd):
    return _forward_impl(hidden_states, ln1_w, ln2_w, Wq, Wk, Wv, Wo,
                         gate_w, Wg, Wu, Wd)


# speedup vs baseline: 1.1245x; 1.0999x over previous
import jax
import jax.numpy as jnp
from jax.experimental import pallas as pl

B, S, H = 1, 2048, 1024
NH, NKH, HD = 16, 4, 64
E, K, F = 8, 2, 512
EPS = 1e-05
THETA = 500000.0

SB = 256
NTB = S // SB
SBQ = 512
SBK = 256
GROUP = NH // NKH
GW = GROUP * HD
SCALE = 0.125

_F32 = jnp.float32
_BF16 = jnp.bfloat16


def _bf(x):
    return x.astype(_BF16)


def _shift_up(x, s):
    return jnp.concatenate([x[:, s:], x[:, :s]], axis=1)


def _shift_dn(x, s):
    return jnp.concatenate([x[:, -s:], x[:, :-s]], axis=1)


def _rope_full(x, cos_t, sina_t, sinb_t):
    half = HD // 2
    return x * cos_t + _shift_up(x, half) * sina_t + _shift_dn(x, half) * sinb_t


def _pre_kernel(h_ref, ln1_ref, wq_ref, wk_ref, wv_ref,
                cos_ref, sina_ref, sinb_ref, q_ref, k_ref, v_ref):
    x = h_ref[...]
    var = jnp.mean(x * x, axis=1, keepdims=True)
    x = _bf(ln1_ref[...] * (x * jax.lax.rsqrt(var + EPS)))
    q = jnp.dot(x, wq_ref[...], preferred_element_type=_F32)
    k = jnp.dot(x, wk_ref[...], preferred_element_type=_F32)
    v = jnp.dot(x, wv_ref[...], preferred_element_type=_F32)
    cos = cos_ref[...]
    sina = sina_ref[...]
    sinb = sinb_ref[...]
    cos_q = jnp.concatenate([cos] * NH, axis=1)
    sina_q = jnp.concatenate([sina] * NH, axis=1)
    sinb_q = jnp.concatenate([sinb] * NH, axis=1)
    q_ref[...] = _bf(_rope_full(q, cos_q, sina_q, sinb_q))
    cos_k = jnp.concatenate([cos] * NKH, axis=1)
    sina_k = jnp.concatenate([sina] * NKH, axis=1)
    sinb_k = jnp.concatenate([sinb] * NKH, axis=1)
    kr = _bf(_rope_full(k, cos_k, sina_k, sinb_k))
    vb = _bf(v)
    onescol = _bf(jax.lax.broadcasted_iota(jnp.int32, (vb.shape[0], HD), 1)
                  == 0)
    for h in range(NKH):
        k_ref[h] = kr[:, h * HD:(h + 1) * HD]
        v_ref[h] = jnp.concatenate([vb[:, h * HD:(h + 1) * HD], onescol],
                                   axis=1)


def _attn_kernel(q_ref, k_ref, v_ref, o_ref):
    qb = pl.program_id(1)
    q4 = q_ref[...]
    qm = jnp.concatenate(
        [q4[:, j * HD:(j + 1) * HD] for j in range(GROUP)], axis=0)
    neg = jnp.finfo(_F32).min
    gsb = GROUP * SBQ

    def step(s, carry, vc):
        acc, m = carry
        m_new = jnp.maximum(m, jnp.max(s, axis=1, keepdims=True))
        alpha = jnp.exp(m - m_new)
        p = jnp.exp(s - m_new)
        acc = acc * alpha + jnp.dot(_bf(p), vc, preferred_element_type=_F32)
        return acc, m_new

    def chunk(c, carry, masked):
        kc = k_ref[0, pl.ds(c * SBK, SBK), :]
        vc = v_ref[0, pl.ds(c * SBK, SBK), :]
        s = jax.lax.dot_general(
            qm, kc, (((1,), (1,)), ((), ())),
            preferred_element_type=_F32) * SCALE
        if masked:
            rows = jax.lax.broadcasted_iota(jnp.int32, s.shape, 0) & (SBQ - 1)
            cols = (c - qb * (SBQ // SBK)) * SBK \
                + jax.lax.broadcasted_iota(jnp.int32, s.shape, 1)
            s = jnp.where(rows >= cols, s, neg)
        return step(s, carry, vc)

    acc0 = jnp.zeros((gsb, 2 * HD), _F32)
    m0 = jnp.full((gsb, 1), neg, _F32)
    nfull = qb * (SBQ // SBK)
    carry = jax.lax.fori_loop(0, nfull, lambda c, cr: chunk(c, cr, False),
                              (acc0, m0))
    for t in range(SBQ // SBK):
        carry = chunk(nfull + t, carry, True)
    acc, m = carry
    o = acc[:, :HD] / acc[:, HD:HD + 1]
    o_ref[...] = _bf(jnp.concatenate(
        [o[j * SBQ:(j + 1) * SBQ, :] for j in range(GROUP)], axis=1))


def _post_kernel(ao_ref, wo_ref, h_ref, ln2_ref, gate_ref,
                 h2_ref, x2_ref, logits_ref):
    attn = jnp.dot(ao_ref[...], wo_ref[...], preferred_element_type=_F32)
    h2 = h_ref[...] + attn
    var = jnp.mean(h2 * h2, axis=1, keepdims=True)
    x2 = ln2_ref[...] * (h2 * jax.lax.rsqrt(var + EPS))
    h2_ref[...] = h2
    x2_ref[...] = _bf(x2)
    logits_ref[...] = jnp.dot(x2, gate_ref[...], preferred_element_type=_F32)


EPG = 2
NEG = E // EPG
SM = 1024


def _moe_kernel(x2_ref, logits_ref, wgu_ref, wd_ref, h2_ref, o_ref):
    eg = pl.program_id(1)
    logits = logits_ref[...]
    col = jax.lax.broadcasted_iota(jnp.int32, logits.shape, 1)
    m1 = jnp.max(logits, axis=1, keepdims=True)
    a1 = jnp.min(jnp.where(logits == m1, col, E), axis=1, keepdims=True)
    masked = jnp.where(col == a1, -jnp.inf, logits)
    m2 = jnp.max(masked, axis=1, keepdims=True)
    a2 = jnp.min(jnp.where(masked == m2, col, E), axis=1, keepdims=True)
    t = jnp.exp(m2 - m1)
    w1 = 1.0 / (1.0 + t)
    w2 = t / (1.0 + t)

    x = x2_ref[...]
    total = None
    for j in range(EPG):
        e = eg * EPG + j
        w_e = jnp.where(a1 == e, w1, 0.0) + jnp.where(a2 == e, w2, 0.0)
        gu = jnp.dot(x, wgu_ref[e], preferred_element_type=_F32)
        g = gu[:, :F]
        u = gu[:, F:]
        act = _bf((g * jax.lax.logistic(g)) * u)
        d = jnp.dot(act, wd_ref[e], preferred_element_type=_F32)
        contrib = w_e * d
        total = contrib if total is None else total + contrib

    @pl.when(eg == 0)
    def _():
        o_ref[...] = h2_ref[...] + total

    @pl.when(eg > 0)
    def _():
        o_ref[...] += total


@jax.jit
def _forward_impl(h3, ln1_w, ln2_w, Wq, Wk, Wv, Wo, gate_w, Wg, Wu, Wd):
    h = h3.reshape(S, H)
    pos = jnp.arange(S, dtype=_F32)
    inv_freq = 1.0 / (THETA ** (jnp.arange(0, HD, 2, dtype=_F32) / HD))
    freqs = pos[:, None] * inv_freq[None, :]
    emb = jnp.concatenate([freqs, freqs], axis=-1)
    cos = jnp.cos(emb)
    sin = jnp.sin(emb)
    half = HD // 2
    lane = jnp.arange(HD)
    sina = jnp.where(lane < half, -sin, 0.0)
    sinb = jnp.where(lane >= half, sin, 0.0)

    q, k, v = pl.pallas_call(
        _pre_kernel,
        grid=(NTB,),
        in_specs=[
            pl.BlockSpec((SB, H), lambda i: (i, 0)),
            pl.BlockSpec((1, H), lambda i: (0, 0)),
            pl.BlockSpec((H, NH * HD), lambda i: (0, 0)),
            pl.BlockSpec((H, NKH * HD), lambda i: (0, 0)),
            pl.BlockSpec((H, NKH * HD), lambda i: (0, 0)),
            pl.BlockSpec((SB, HD), lambda i: (i, 0)),
            pl.BlockSpec((SB, HD), lambda i: (i, 0)),
            pl.BlockSpec((SB, HD), lambda i: (i, 0)),
        ],
        out_specs=[
            pl.BlockSpec((SB, NH * HD), lambda i: (i, 0)),
            pl.BlockSpec((NKH, SB, HD), lambda i: (0, i, 0)),
            pl.BlockSpec((NKH, SB, 2 * HD), lambda i: (0, i, 0)),
        ],
        out_shape=[
            jax.ShapeDtypeStruct((S, NH * HD), _BF16),
            jax.ShapeDtypeStruct((NKH, S, HD), _BF16),
            jax.ShapeDtypeStruct((NKH, S, 2 * HD), _BF16),
        ],
    )(h, ln1_w.reshape(1, H), _bf(Wq), _bf(Wk), _bf(Wv), cos, sina, sinb)

    o = pl.pallas_call(
        _attn_kernel,
        grid=(NKH, S // SBQ),
        in_specs=[
            pl.BlockSpec((SBQ, GW), lambda g, i: (i, g)),
            pl.BlockSpec((1, S, HD), lambda g, i: (g, 0, 0)),
            pl.BlockSpec((1, S, 2 * HD), lambda g, i: (g, 0, 0)),
        ],
        out_specs=pl.BlockSpec((SBQ, GW), lambda g, i: (i, g)),
        out_shape=jax.ShapeDtypeStruct((S, NH * HD), _BF16),
    )(q, k, v)

    h2, x2, logits = pl.pallas_call(
        _post_kernel,
        grid=(NTB,),
        in_specs=[
            pl.BlockSpec((SB, NH * HD), lambda i: (i, 0)),
            pl.BlockSpec((NH * HD, H), lambda i: (0, 0)),
            pl.BlockSpec((SB, H), lambda i: (i, 0)),
            pl.BlockSpec((1, H), lambda i: (0, 0)),
            pl.BlockSpec((H, E), lambda i: (0, 0)),
        ],
        out_specs=[
            pl.BlockSpec((SB, H), lambda i: (i, 0)),
            pl.BlockSpec((SB, H), lambda i: (i, 0)),
            pl.BlockSpec((SB, E), lambda i: (i, 0)),
        ],
        out_shape=[
            jax.ShapeDtypeStruct((S, H), _F32),
            jax.ShapeDtypeStruct((S, H), _BF16),
            jax.ShapeDtypeStruct((S, E), _F32),
        ],
    )(o, _bf(Wo), h, ln2_w.reshape(1, H), gate_w)

    out = pl.pallas_call(
        _moe_kernel,
        grid=(S // SM, NEG),
        in_specs=[
            pl.BlockSpec((SM, H), lambda t, e: (t, 0)),
            pl.BlockSpec((SM, E), lambda t, e: (t, 0)),
            pl.BlockSpec((E, H, 2 * F), lambda t, e: (0, 0, 0)),
            pl.BlockSpec((E, F, H), lambda t, e: (0, 0, 0)),
            pl.BlockSpec((SM, H), lambda t, e: (t, 0)),
        ],
        out_specs=pl.BlockSpec((SM, H), lambda t, e: (t, 0)),
        out_shape=jax.ShapeDtypeStruct((S, H), _F32),
    )(x2, logits, _bf(jnp.concatenate([Wg, Wu], axis=2)), _bf(Wd), h2)

    return out.reshape(B, S, H)


def kernel(hidden_states, ln1_w, ln2_w, Wq, Wk, Wv, Wo, gate_w, Wg, Wu, Wd):
    return _forward_impl(hidden_states, ln1_w, ln2_w, Wq, Wk, Wv, Wo,
                         gate_w, Wg, Wu, Wd)


# speedup vs baseline: 1.1871x; 1.0557x over previous
import jax
import jax.numpy as jnp
from jax.experimental import pallas as pl
from jax.experimental.pallas import tpu as pltpu

B, S, H = 1, 2048, 1024
NH, NKH, HD = 16, 4, 64
E, K, F = 8, 2, 512
EPS = 1e-05
THETA = 500000.0

SBQ = 512
NQB = S // SBQ
SBK = 256
GROUP = NH // NKH
GW = GROUP * HD
SCALE = 0.125

EPG = 2
NEG = E // EPG
SM = 1024

_F32 = jnp.float32
_BF16 = jnp.bfloat16


def _bf(x):
    return x.astype(_BF16)


def _shift_up(x, s):
    return jnp.concatenate([x[:, s:], x[:, :s]], axis=1)


def _shift_dn(x, s):
    return jnp.concatenate([x[:, -s:], x[:, :-s]], axis=1)


def _rope_full(x, cos_t, sina_t, sinb_t):
    half = HD // 2
    return x * cos_t + _shift_up(x, half) * sina_t + _shift_dn(x, half) * sinb_t


def _layer_kernel(h_ref, ln1_ref, wq_ref, wk_ref, wv_ref,
                  cos_ref, sina_ref, sinb_ref, wo_ref, ln2_ref, gate_ref,
                  h2_ref, x2_ref, logits_ref, ks_ref, vs_ref):
    i = pl.program_id(0)
    hs = h_ref[...]
    var = jnp.mean(hs * hs, axis=1, keepdims=True)
    xb = _bf(ln1_ref[...] * (hs * jax.lax.rsqrt(var + EPS)))
    q = jnp.dot(xb, wq_ref[...], preferred_element_type=_F32)
    k = jnp.dot(xb, wk_ref[...], preferred_element_type=_F32)
    v = jnp.dot(xb, wv_ref[...], preferred_element_type=_F32)
    cos = cos_ref[...]
    sina = sina_ref[...]
    sinb = sinb_ref[...]
    qr = _bf(_rope_full(q, jnp.concatenate([cos] * NH, axis=1),
                        jnp.concatenate([sina] * NH, axis=1),
                        jnp.concatenate([sinb] * NH, axis=1)))
    kr = _bf(_rope_full(k, jnp.concatenate([cos] * NKH, axis=1),
                        jnp.concatenate([sina] * NKH, axis=1),
                        jnp.concatenate([sinb] * NKH, axis=1)))
    vb = _bf(v)
    onescol = _bf(jax.lax.broadcasted_iota(jnp.int32, (SBQ, HD), 1) == 0)
    base = i * SBQ
    for hh in range(NKH):
        ks_ref[hh, pl.ds(base, SBQ), :] = kr[:, hh * HD:(hh + 1) * HD]
        vs_ref[hh, pl.ds(base, SBQ), :] = jnp.concatenate(
            [vb[:, hh * HD:(hh + 1) * HD], onescol], axis=1)

    neg = jnp.finfo(_F32).min
    gsb = GROUP * SBQ
    nfull = i * (SBQ // SBK)

    o_parts = []
    for g in range(NKH):
        qm = jnp.concatenate(
            [qr[:, g * GW + j * HD:g * GW + (j + 1) * HD]
             for j in range(GROUP)], axis=0)

        def step(s, carry, vc):
            acc, m = carry
            m_new = jnp.maximum(m, jnp.max(s, axis=1, keepdims=True))
            alpha = jnp.exp(m - m_new)
            p = jnp.exp(s - m_new)
            acc = acc * alpha + jnp.dot(_bf(p), vc,
                                        preferred_element_type=_F32)
            return acc, m_new

        def chunk(c, carry, masked):
            kc = ks_ref[g, pl.ds(c * SBK, SBK), :]
            vc = vs_ref[g, pl.ds(c * SBK, SBK), :]
            s = jax.lax.dot_general(
                qm, kc, (((1,), (1,)), ((), ())),
                preferred_element_type=_F32) * SCALE
            if masked:
                rows = (jax.lax.broadcasted_iota(jnp.int32, s.shape, 0)
                        & (SBQ - 1))
                cols = (c - nfull) * SBK \
                    + jax.lax.broadcasted_iota(jnp.int32, s.shape, 1)
                s = jnp.where(rows >= cols, s, neg)
            return step(s, carry, vc)

        acc0 = jnp.zeros((gsb, 2 * HD), _F32)
        m0 = jnp.full((gsb, 1), neg, _F32)
        carry = jax.lax.fori_loop(0, nfull,
                                  lambda c, cr: chunk(c, cr, False),
                                  (acc0, m0))
        for t in range(SBQ // SBK):
            carry = chunk(nfull + t, carry, True)
        acc, _ = carry
        og = acc[:, :HD] / acc[:, HD:HD + 1]
        o_parts.extend(og[j * SBQ:(j + 1) * SBQ, :] for j in range(GROUP))

    o = _bf(jnp.concatenate(o_parts, axis=1))
    attn = jnp.dot(o, wo_ref[...], preferred_element_type=_F32)
    h2 = hs + attn
    var2 = jnp.mean(h2 * h2, axis=1, keepdims=True)
    x2 = ln2_ref[...] * (h2 * jax.lax.rsqrt(var2 + EPS))
    h2_ref[...] = h2
    x2_ref[...] = _bf(x2)
    logits_ref[...] = jnp.dot(x2, gate_ref[...], preferred_element_type=_F32)


def _moe_kernel(x2_ref, logits_ref, wgu_ref, wd_ref, h2_ref, o_ref):
    eg = pl.program_id(1)
    logits = logits_ref[...]
    col = jax.lax.broadcasted_iota(jnp.int32, logits.shape, 1)
    m1 = jnp.max(logits, axis=1, keepdims=True)
    a1 = jnp.min(jnp.where(logits == m1, col, E), axis=1, keepdims=True)
    masked = jnp.where(col == a1, -jnp.inf, logits)
    m2 = jnp.max(masked, axis=1, keepdims=True)
    a2 = jnp.min(jnp.where(masked == m2, col, E), axis=1, keepdims=True)
    t = jnp.exp(m2 - m1)
    w1 = 1.0 / (1.0 + t)
    w2 = t / (1.0 + t)

    x = x2_ref[...]
    total = None
    for j in range(EPG):
        e = eg * EPG + j
        w_e = jnp.where(a1 == e, w1, 0.0) + jnp.where(a2 == e, w2, 0.0)
        gu = jnp.dot(x, wgu_ref[e], preferred_element_type=_F32)
        g = gu[:, :F]
        u = gu[:, F:]
        act = _bf((g * jax.lax.logistic(g)) * u)
        d = jnp.dot(act, wd_ref[e], preferred_element_type=_F32)
        contrib = w_e * d
        total = contrib if total is None else total + contrib

    @pl.when(eg == 0)
    def _():
        o_ref[...] = h2_ref[...] + total

    @pl.when(eg > 0)
    def _():
        o_ref[...] += total


@jax.jit
def _forward_impl(h3, ln1_w, ln2_w, Wq, Wk, Wv, Wo, gate_w, Wg, Wu, Wd):
    h = h3.reshape(S, H)
    pos = jnp.arange(S, dtype=_F32)
    inv_freq = 1.0 / (THETA ** (jnp.arange(0, HD, 2, dtype=_F32) / HD))
    freqs = pos[:, None] * inv_freq[None, :]
    emb = jnp.concatenate([freqs, freqs], axis=-1)
    cos = jnp.cos(emb)
    sin = jnp.sin(emb)
    half = HD // 2
    lane = jnp.arange(HD)
    sina = jnp.where(lane < half, -sin, 0.0)
    sinb = jnp.where(lane >= half, sin, 0.0)

    h2, x2, logits = pl.pallas_call(
        _layer_kernel,
        grid=(NQB,),
        in_specs=[
            pl.BlockSpec((SBQ, H), lambda i: (i, 0)),
            pl.BlockSpec((1, H), lambda i: (0, 0)),
            pl.BlockSpec((H, NH * HD), lambda i: (0, 0)),
            pl.BlockSpec((H, NKH * HD), lambda i: (0, 0)),
            pl.BlockSpec((H, NKH * HD), lambda i: (0, 0)),
            pl.BlockSpec((SBQ, HD), lambda i: (i, 0)),
            pl.BlockSpec((SBQ, HD), lambda i: (i, 0)),
            pl.BlockSpec((SBQ, HD), lambda i: (i, 0)),
            pl.BlockSpec((NH * HD, H), lambda i: (0, 0)),
            pl.BlockSpec((1, H), lambda i: (0, 0)),
            pl.BlockSpec((H, E), lambda i: (0, 0)),
        ],
        out_specs=[
            pl.BlockSpec((SBQ, H), lambda i: (i, 0)),
            pl.BlockSpec((SBQ, H), lambda i: (i, 0)),
            pl.BlockSpec((SBQ, E), lambda i: (i, 0)),
        ],
        out_shape=[
            jax.ShapeDtypeStruct((S, H), _F32),
            jax.ShapeDtypeStruct((S, H), _BF16),
            jax.ShapeDtypeStruct((S, E), _F32),
        ],
        scratch_shapes=[
            pltpu.VMEM((NKH, S, HD), _BF16),
            pltpu.VMEM((NKH, S, 2 * HD), _BF16),
        ],
    )(h, ln1_w.reshape(1, H), _bf(Wq), _bf(Wk), _bf(Wv), cos, sina, sinb,
      _bf(Wo), ln2_w.reshape(1, H), gate_w)

    out = pl.pallas_call(
        _moe_kernel,
        grid=(S // SM, NEG),
        in_specs=[
            pl.BlockSpec((SM, H), lambda t, e: (t, 0)),
            pl.BlockSpec((SM, E), lambda t, e: (t, 0)),
            pl.BlockSpec((E, H, 2 * F), lambda t, e: (0, 0, 0)),
            pl.BlockSpec((E, F, H), lambda t, e: (0, 0, 0)),
            pl.BlockSpec((SM, H), lambda t, e: (t, 0)),
        ],
        out_specs=pl.BlockSpec((SM, H), lambda t, e: (t, 0)),
        out_shape=jax.ShapeDtypeStruct((S, H), _F32),
    )(x2, logits, _bf(jnp.concatenate([Wg, Wu], axis=2)), _bf(Wd), h2)

    return out.reshape(B, S, H)


def kernel(hidden_states, ln1_w, ln2_w, Wq, Wk, Wv, Wo, gate_w, Wg, Wu, Wd):
    return _forward_impl(hidden_states, ln1_w, ln2_w, Wq, Wk, Wv, Wo,
                         gate_w, Wg, Wu, Wd)


# speedup vs baseline: 1.3415x; 1.1300x over previous
import jax
import jax.numpy as jnp
from jax.experimental import pallas as pl
from jax.experimental.pallas import tpu as pltpu

B, S, H = 1, 2048, 1024
NH, NKH, HD = 16, 4, 64
E, K, F = 8, 2, 512
EPS = 1e-05
THETA = 500000.0

SBQ = 512
NQB = S // SBQ
SBK = 512
GROUP = NH // NKH
GW = GROUP * HD
SCALE = 0.125

EPG = 2
NEG = E // EPG
SM = 1024

_F32 = jnp.float32
_BF16 = jnp.bfloat16


def _bf(x):
    return x.astype(_BF16)


def _shift_up(x, s):
    return jnp.concatenate([x[:, s:], x[:, :s]], axis=1)


def _shift_dn(x, s):
    return jnp.concatenate([x[:, -s:], x[:, :-s]], axis=1)


def _rope_full(x, cos_t, sina_t, sinb_t):
    half = HD // 2
    return x * cos_t + _shift_up(x, half) * sina_t + _shift_dn(x, half) * sinb_t


def _layer_kernel(h_ref, ln1_ref, wq_ref, wk_ref, wv_ref,
                  cos_ref, sina_ref, sinb_ref, wo_ref, ln2_ref, gate_ref,
                  h2_ref, x2_ref, logits_ref, ks_ref, vs_ref):
    i = pl.program_id(0)
    hs = h_ref[...]
    var = jnp.mean(hs * hs, axis=1, keepdims=True)
    xb = _bf(ln1_ref[...] * (hs * jax.lax.rsqrt(var + EPS)))
    q = jnp.dot(xb, wq_ref[...], preferred_element_type=_F32)
    k = jnp.dot(xb, wk_ref[...], preferred_element_type=_F32)
    v = jnp.dot(xb, wv_ref[...], preferred_element_type=_F32)
    cos = cos_ref[...]
    sina = sina_ref[...]
    sinb = sinb_ref[...]
    qr = _bf(_rope_full(q, jnp.concatenate([cos] * NH, axis=1),
                        jnp.concatenate([sina] * NH, axis=1),
                        jnp.concatenate([sinb] * NH, axis=1)))
    kr = _bf(_rope_full(k, jnp.concatenate([cos] * NKH, axis=1),
                        jnp.concatenate([sina] * NKH, axis=1),
                        jnp.concatenate([sinb] * NKH, axis=1)))
    vb = _bf(v)
    onescol = _bf(jax.lax.broadcasted_iota(jnp.int32, (SBQ, HD), 1) == 0)
    base = i * SBQ
    for hh in range(NKH):
        ks_ref[hh, pl.ds(base, SBQ), :] = kr[:, hh * HD:(hh + 1) * HD]
        vs_ref[hh, pl.ds(base, SBQ), :] = jnp.concatenate(
            [vb[:, hh * HD:(hh + 1) * HD], onescol], axis=1)

    neg = jnp.finfo(_F32).min
    gsb = GROUP * SBQ
    nfull = i * (SBQ // SBK)

    o_parts = []
    for g in range(NKH):
        qm = jnp.concatenate(
            [qr[:, g * GW + j * HD:g * GW + (j + 1) * HD]
             for j in range(GROUP)], axis=0)

        def step(s, carry, vc):
            acc, m = carry
            m_new = jnp.maximum(m, jnp.max(s, axis=1, keepdims=True))
            alpha = jnp.exp(m - m_new)
            p = jnp.exp(s - m_new)
            acc = acc * alpha + jnp.dot(_bf(p), vc,
                                        preferred_element_type=_F32)
            return acc, m_new

        def chunk(c, carry, masked):
            kc = ks_ref[g, pl.ds(c * SBK, SBK), :]
            vc = vs_ref[g, pl.ds(c * SBK, SBK), :]
            s = jax.lax.dot_general(
                qm, kc, (((1,), (1,)), ((), ())),
                preferred_element_type=_F32) * SCALE
            if masked:
                rows = (jax.lax.broadcasted_iota(jnp.int32, s.shape, 0)
                        & (SBQ - 1))
                cols = (c - nfull) * SBK \
                    + jax.lax.broadcasted_iota(jnp.int32, s.shape, 1)
                s = jnp.where(rows >= cols, s, neg)
            return step(s, carry, vc)

        acc0 = jnp.zeros((gsb, 2 * HD), _F32)
        m0 = jnp.full((gsb, 1), neg, _F32)
        carry = jax.lax.fori_loop(0, nfull,
                                  lambda c, cr: chunk(c, cr, False),
                                  (acc0, m0))
        for t in range(SBQ // SBK):
            carry = chunk(nfull + t, carry, True)
        acc, _ = carry
        og = acc[:, :HD] / acc[:, HD:HD + 1]
        o_parts.extend(og[j * SBQ:(j + 1) * SBQ, :] for j in range(GROUP))

    o = _bf(jnp.concatenate(o_parts, axis=1))
    attn = jnp.dot(o, wo_ref[...], preferred_element_type=_F32)
    h2 = hs + attn
    var2 = jnp.mean(h2 * h2, axis=1, keepdims=True)
    x2 = ln2_ref[...] * (h2 * jax.lax.rsqrt(var2 + EPS))
    h2_ref[...] = h2
    x2_ref[...] = _bf(x2)
    logits_ref[...] = jnp.dot(x2, gate_ref[...], preferred_element_type=_F32)


def _moe_kernel(x2_ref, logits_ref, wgu_ref, wd_ref, h2_ref, o_ref):
    eg = pl.program_id(1)
    logits = logits_ref[...]
    col = jax.lax.broadcasted_iota(jnp.int32, logits.shape, 1)
    m1 = jnp.max(logits, axis=1, keepdims=True)
    a1 = jnp.min(jnp.where(logits == m1, col, E), axis=1, keepdims=True)
    masked = jnp.where(col == a1, -jnp.inf, logits)
    m2 = jnp.max(masked, axis=1, keepdims=True)
    a2 = jnp.min(jnp.where(masked == m2, col, E), axis=1, keepdims=True)
    t = jnp.exp(m2 - m1)
    w1 = 1.0 / (1.0 + t)
    w2 = t / (1.0 + t)

    x = x2_ref[...]
    total = None
    for j in range(EPG):
        e = eg * EPG + j
        w_e = jnp.where(a1 == e, w1, 0.0) + jnp.where(a2 == e, w2, 0.0)
        gu = jnp.dot(x, wgu_ref[e], preferred_element_type=_F32)
        g = gu[:, :F]
        u = gu[:, F:]
        act = _bf((g * jax.lax.logistic(g)) * u)
        d = jnp.dot(act, wd_ref[e], preferred_element_type=_F32)
        contrib = w_e * d
        total = contrib if total is None else total + contrib

    @pl.when(eg == 0)
    def _():
        o_ref[...] = h2_ref[...] + total

    @pl.when(eg > 0)
    def _():
        o_ref[...] += total


@jax.jit
def _forward_impl(h3, ln1_w, ln2_w, Wq, Wk, Wv, Wo, gate_w, Wg, Wu, Wd):
    h = h3.reshape(S, H)
    pos = jnp.arange(S, dtype=_F32)
    inv_freq = 1.0 / (THETA ** (jnp.arange(0, HD, 2, dtype=_F32) / HD))
    freqs = pos[:, None] * inv_freq[None, :]
    emb = jnp.concatenate([freqs, freqs], axis=-1)
    cos = jnp.cos(emb)
    sin = jnp.sin(emb)
    half = HD // 2
    lane = jnp.arange(HD)
    sina = jnp.where(lane < half, -sin, 0.0)
    sinb = jnp.where(lane >= half, sin, 0.0)

    h2, x2, logits = pl.pallas_call(
        _layer_kernel,
        grid=(NQB,),
        in_specs=[
            pl.BlockSpec((SBQ, H), lambda i: (i, 0)),
            pl.BlockSpec((1, H), lambda i: (0, 0)),
            pl.BlockSpec((H, NH * HD), lambda i: (0, 0)),
            pl.BlockSpec((H, NKH * HD), lambda i: (0, 0)),
            pl.BlockSpec((H, NKH * HD), lambda i: (0, 0)),
            pl.BlockSpec((SBQ, HD), lambda i: (i, 0)),
            pl.BlockSpec((SBQ, HD), lambda i: (i, 0)),
            pl.BlockSpec((SBQ, HD), lambda i: (i, 0)),
            pl.BlockSpec((NH * HD, H), lambda i: (0, 0)),
            pl.BlockSpec((1, H), lambda i: (0, 0)),
            pl.BlockSpec((H, E), lambda i: (0, 0)),
        ],
        out_specs=[
            pl.BlockSpec((SBQ, H), lambda i: (i, 0)),
            pl.BlockSpec((SBQ, H), lambda i: (i, 0)),
            pl.BlockSpec((SBQ, E), lambda i: (i, 0)),
        ],
        out_shape=[
            jax.ShapeDtypeStruct((S, H), _F32),
            jax.ShapeDtypeStruct((S, H), _BF16),
            jax.ShapeDtypeStruct((S, E), _F32),
        ],
        scratch_shapes=[
            pltpu.VMEM((NKH, S, HD), _BF16),
            pltpu.VMEM((NKH, S, 2 * HD), _BF16),
        ],
    )(h, ln1_w.reshape(1, H), _bf(Wq), _bf(Wk), _bf(Wv), cos, sina, sinb,
      _bf(Wo), ln2_w.reshape(1, H), gate_w)

    out = pl.pallas_call(
        _moe_kernel,
        grid=(S // SM, NEG),
        in_specs=[
            pl.BlockSpec((SM, H), lambda t, e: (t, 0)),
            pl.BlockSpec((SM, E), lambda t, e: (t, 0)),
            pl.BlockSpec((E, H, 2 * F), lambda t, e: (0, 0, 0)),
            pl.BlockSpec((E, F, H), lambda t, e: (0, 0, 0)),
            pl.BlockSpec((SM, H), lambda t, e: (t, 0)),
        ],
        out_specs=pl.BlockSpec((SM, H), lambda t, e: (t, 0)),
        out_shape=jax.ShapeDtypeStruct((S, H), _F32),
    )(x2, logits, _bf(jnp.concatenate([Wg, Wu], axis=2)), _bf(Wd), h2)

    return out.reshape(B, S, H)


def kernel(hidden_states, ln1_w, ln2_w, Wq, Wk, Wv, Wo, gate_w, Wg, Wu, Wd):
    return _forward_impl(hidden_states, ln1_w, ln2_w, Wq, Wk, Wv, Wo,
                         gate_w, Wg, Wu, Wd)


# speedup vs baseline: 1.3608x; 1.0144x over previous
import jax
import jax.numpy as jnp
from jax.experimental import pallas as pl
from jax.experimental.pallas import tpu as pltpu

B, S, H = 1, 2048, 1024
NH, NKH, HD = 16, 4, 64
E, K, F = 8, 2, 512
EPS = 1e-05
THETA = 500000.0

SBQ = 512
NQB = S // SBQ
SBK = 512
GROUP = NH // NKH
GW = GROUP * HD
SCALE = 0.125

EPG = 4
NEG = E // EPG
SM = 1024

_F32 = jnp.float32
_BF16 = jnp.bfloat16


def _bf(x):
    return x.astype(_BF16)


def _shift_up(x, s):
    return jnp.concatenate([x[:, s:], x[:, :s]], axis=1)


def _shift_dn(x, s):
    return jnp.concatenate([x[:, -s:], x[:, :-s]], axis=1)


def _rope_full(x, cos_t, sina_t, sinb_t):
    half = HD // 2
    return x * cos_t + _shift_up(x, half) * sina_t + _shift_dn(x, half) * sinb_t


def _layer_kernel(h_ref, ln1_ref, wq_ref, wk_ref, wv_ref,
                  cos_ref, sina_ref, sinb_ref, wo_ref, ln2_ref, gate_ref,
                  h2_ref, x2_ref, logits_ref, ks_ref, vs_ref):
    i = pl.program_id(0)
    hs = h_ref[...]
    var = jnp.mean(hs * hs, axis=1, keepdims=True)
    xb = _bf(ln1_ref[...] * (hs * jax.lax.rsqrt(var + EPS)))
    q = jnp.dot(xb, wq_ref[...], preferred_element_type=_F32)
    k = jnp.dot(xb, wk_ref[...], preferred_element_type=_F32)
    v = jnp.dot(xb, wv_ref[...], preferred_element_type=_F32)
    cos = cos_ref[...]
    sina = sina_ref[...]
    sinb = sinb_ref[...]
    qr = _bf(_rope_full(q, jnp.concatenate([cos] * NH, axis=1),
                        jnp.concatenate([sina] * NH, axis=1),
                        jnp.concatenate([sinb] * NH, axis=1)))
    kr = _bf(_rope_full(k, jnp.concatenate([cos] * NKH, axis=1),
                        jnp.concatenate([sina] * NKH, axis=1),
                        jnp.concatenate([sinb] * NKH, axis=1)))
    vb = _bf(v)
    onescol = _bf(jax.lax.broadcasted_iota(jnp.int32, (SBQ, HD), 1) == 0)
    base = i * SBQ
    for hh in range(NKH):
        ks_ref[hh, pl.ds(base, SBQ), :] = kr[:, hh * HD:(hh + 1) * HD]
        vs_ref[hh, pl.ds(base, SBQ), :] = jnp.concatenate(
            [vb[:, hh * HD:(hh + 1) * HD], onescol], axis=1)

    neg = jnp.finfo(_F32).min
    gsb = GROUP * SBQ
    nfull = i * (SBQ // SBK)

    o_parts = []
    for g in range(NKH):
        qm = jnp.concatenate(
            [qr[:, g * GW + j * HD:g * GW + (j + 1) * HD]
             for j in range(GROUP)], axis=0)

        def step(s, carry, vc):
            acc, m = carry
            m_new = jnp.maximum(m, jnp.max(s, axis=1, keepdims=True))
            alpha = jnp.exp(m - m_new)
            p = jnp.exp(s - m_new)
            acc = acc * alpha + jnp.dot(_bf(p), vc,
                                        preferred_element_type=_F32)
            return acc, m_new

        def chunk(c, carry, masked):
            kc = ks_ref[g, pl.ds(c * SBK, SBK), :]
            vc = vs_ref[g, pl.ds(c * SBK, SBK), :]
            s = jax.lax.dot_general(
                qm, kc, (((1,), (1,)), ((), ())),
                preferred_element_type=_F32) * SCALE
            if masked:
                rows = (jax.lax.broadcasted_iota(jnp.int32, s.shape, 0)
                        & (SBQ - 1))
                cols = (c - nfull) * SBK \
                    + jax.lax.broadcasted_iota(jnp.int32, s.shape, 1)
                s = jnp.where(rows >= cols, s, neg)
            return step(s, carry, vc)

        acc0 = jnp.zeros((gsb, 2 * HD), _F32)
        m0 = jnp.full((gsb, 1), neg, _F32)
        carry = jax.lax.fori_loop(0, nfull,
                                  lambda c, cr: chunk(c, cr, False),
                                  (acc0, m0))
        for t in range(SBQ // SBK):
            carry = chunk(nfull + t, carry, True)
        acc, _ = carry
        og = acc[:, :HD] / acc[:, HD:HD + 1]
        o_parts.extend(og[j * SBQ:(j + 1) * SBQ, :] for j in range(GROUP))

    o = _bf(jnp.concatenate(o_parts, axis=1))
    attn = jnp.dot(o, wo_ref[...], preferred_element_type=_F32)
    h2 = hs + attn
    var2 = jnp.mean(h2 * h2, axis=1, keepdims=True)
    x2 = ln2_ref[...] * (h2 * jax.lax.rsqrt(var2 + EPS))
    h2_ref[...] = h2
    x2_ref[...] = _bf(x2)
    logits_ref[...] = jnp.dot(x2, gate_ref[...], preferred_element_type=_F32)


def _moe_kernel(x2_ref, logits_ref, wgu_ref, wd_ref, h2_ref, o_ref):
    eg = pl.program_id(1)
    logits = logits_ref[...]
    col = jax.lax.broadcasted_iota(jnp.int32, logits.shape, 1)
    m1 = jnp.max(logits, axis=1, keepdims=True)
    a1 = jnp.min(jnp.where(logits == m1, col, E), axis=1, keepdims=True)
    masked = jnp.where(col == a1, -jnp.inf, logits)
    m2 = jnp.max(masked, axis=1, keepdims=True)
    a2 = jnp.min(jnp.where(masked == m2, col, E), axis=1, keepdims=True)
    t = jnp.exp(m2 - m1)
    w1 = 1.0 / (1.0 + t)
    w2 = t / (1.0 + t)

    x = x2_ref[...]
    total = None
    for j in range(EPG):
        e = eg * EPG + j
        w_e = jnp.where(a1 == e, w1, 0.0) + jnp.where(a2 == e, w2, 0.0)
        gu = jnp.dot(x, wgu_ref[e], preferred_element_type=_F32)
        g = gu[:, :F]
        u = gu[:, F:]
        act = _bf((g * jax.lax.logistic(g)) * u)
        d = jnp.dot(act, wd_ref[e], preferred_element_type=_F32)
        contrib = w_e * d
        total = contrib if total is None else total + contrib

    @pl.when(eg == 0)
    def _():
        o_ref[...] = h2_ref[...] + total

    @pl.when(eg > 0)
    def _():
        o_ref[...] += total


@jax.jit
def _forward_impl(h3, ln1_w, ln2_w, Wq, Wk, Wv, Wo, gate_w, Wg, Wu, Wd):
    h = h3.reshape(S, H)
    pos = jnp.arange(S, dtype=_F32)
    inv_freq = 1.0 / (THETA ** (jnp.arange(0, HD, 2, dtype=_F32) / HD))
    freqs = pos[:, None] * inv_freq[None, :]
    emb = jnp.concatenate([freqs, freqs], axis=-1)
    cos = jnp.cos(emb)
    sin = jnp.sin(emb)
    half = HD // 2
    lane = jnp.arange(HD)
    sina = jnp.where(lane < half, -sin, 0.0)
    sinb = jnp.where(lane >= half, sin, 0.0)

    h2, x2, logits = pl.pallas_call(
        _layer_kernel,
        grid=(NQB,),
        in_specs=[
            pl.BlockSpec((SBQ, H), lambda i: (i, 0)),
            pl.BlockSpec((1, H), lambda i: (0, 0)),
            pl.BlockSpec((H, NH * HD), lambda i: (0, 0)),
            pl.BlockSpec((H, NKH * HD), lambda i: (0, 0)),
            pl.BlockSpec((H, NKH * HD), lambda i: (0, 0)),
            pl.BlockSpec((SBQ, HD), lambda i: (i, 0)),
            pl.BlockSpec((SBQ, HD), lambda i: (i, 0)),
            pl.BlockSpec((SBQ, HD), lambda i: (i, 0)),
            pl.BlockSpec((NH * HD, H), lambda i: (0, 0)),
            pl.BlockSpec((1, H), lambda i: (0, 0)),
            pl.BlockSpec((H, E), lambda i: (0, 0)),
        ],
        out_specs=[
            pl.BlockSpec((SBQ, H), lambda i: (i, 0)),
            pl.BlockSpec((SBQ, H), lambda i: (i, 0)),
            pl.BlockSpec((SBQ, E), lambda i: (i, 0)),
        ],
        out_shape=[
            jax.ShapeDtypeStruct((S, H), _F32),
            jax.ShapeDtypeStruct((S, H), _BF16),
            jax.ShapeDtypeStruct((S, E), _F32),
        ],
        scratch_shapes=[
            pltpu.VMEM((NKH, S, HD), _BF16),
            pltpu.VMEM((NKH, S, 2 * HD), _BF16),
        ],
    )(h, ln1_w.reshape(1, H), _bf(Wq), _bf(Wk), _bf(Wv), cos, sina, sinb,
      _bf(Wo), ln2_w.reshape(1, H), gate_w)

    out = pl.pallas_call(
        _moe_kernel,
        grid=(S // SM, NEG),
        in_specs=[
            pl.BlockSpec((SM, H), lambda t, e: (t, 0)),
            pl.BlockSpec((SM, E), lambda t, e: (t, 0)),
            pl.BlockSpec((E, H, 2 * F), lambda t, e: (0, 0, 0)),
            pl.BlockSpec((E, F, H), lambda t, e: (0, 0, 0)),
            pl.BlockSpec((SM, H), lambda t, e: (t, 0)),
        ],
        out_specs=pl.BlockSpec((SM, H), lambda t, e: (t, 0)),
        out_shape=jax.ShapeDtypeStruct((S, H), _F32),
    )(x2, logits, _bf(jnp.concatenate([Wg, Wu], axis=2)), _bf(Wd), h2)

    return out.reshape(B, S, H)


def kernel(hidden_states, ln1_w, ln2_w, Wq, Wk, Wv, Wo, gate_w, Wg, Wu, Wd):
    return _forward_impl(hidden_states, ln1_w, ln2_w, Wq, Wk, Wv, Wo,
                         gate_w, Wg, Wu, Wd)
